# baseline (speedup 1.0000x reference)
import jax
import jax.numpy as jnp
from jax.experimental import pallas as pl
from jax.experimental.pallas import tpu as pltpu


def _passthrough_copy(wav_ref, len_ref, wav_out, len_out, wav_sem, len_sem):
    wav_cp = pltpu.make_async_copy(wav_ref, wav_out, wav_sem)
    len_cp = pltpu.make_async_copy(len_ref, len_out, len_sem)
    wav_cp.start()
    len_cp.start()
    wav_cp.wait()
    len_cp.wait()


def kernel(wav_batch, lengths):
    lengths = jnp.asarray(lengths).astype(jnp.int32)
    hbm = pl.BlockSpec(memory_space=pltpu.MemorySpace.HBM)
    padded_features, feature_lengths = pl.pallas_call(
        _passthrough_copy,
        out_shape=(
            jax.ShapeDtypeStruct(wav_batch.shape, wav_batch.dtype),
            jax.ShapeDtypeStruct(lengths.shape, jnp.int32),
        ),
        in_specs=[hbm, hbm],
        out_specs=(hbm, hbm),
        scratch_shapes=[pltpu.SemaphoreType.DMA, pltpu.SemaphoreType.DMA],
    )(wav_batch, lengths)
    return (padded_features, feature_lengths)
